# baseline (speedup 1.0000x reference)
import functools

import jax
import jax.numpy as jnp
from jax import lax
from jax.experimental import pallas as pl
from jax.experimental.pallas import tpu as pltpu
from jax.experimental.pallas import tpu_sc as plsc

_NC = 2
_NS = 16
_NW = _NC * _NS
_CH = 128
_VC = 32768
_HC = _VC // 2
_BT = 512


def _repack_body(tT, out):
    x = tT[...]
    x2 = jnp.concatenate([x[:, :_HC], x[:, _HC:]], axis=0)
    out[...] = jnp.transpose(x2)


def _repack(tableT, V, D):
    nb = pl.cdiv(V, _VC)
    return pl.pallas_call(
        _repack_body,
        grid=(nb,),
        in_specs=[pl.BlockSpec((D, _VC), lambda i: (0, i))],
        out_specs=pl.BlockSpec((_HC, 2 * D), lambda i: (i, 0)),
        out_shape=jax.ShapeDtypeStruct((nb * _HC, 2 * D), jnp.float32),
        compiler_params=pltpu.CompilerParams(
            dimension_semantics=("parallel",)),
    )(tableT)


def _sc_gather(emb2, idlists, D):
    W = 2 * D
    cpws = [ids.shape[1] for ids in idlists]
    buf_rows = 640

    mesh = plsc.VectorSubcoreMesh(core_axis_name="c", subcore_axis_name="s")

    @functools.partial(
        pl.kernel,
        mesh=mesh,
        out_type=[
            jax.ShapeDtypeStruct((_NW * cpw * _CH, W), jnp.float32)
            for cpw in cpws
        ],
        scratch_types=[
            *[pltpu.VMEM((cpw, _CH), jnp.int32) for cpw in cpws],
            pltpu.VMEM((buf_rows, W), jnp.float32),
            pltpu.SemaphoreType.DMA,
        ],
    )
    def gather(emb, *refs):
        ids_refs = refs[:len(cpws)]
        out_refs = refs[len(cpws):2 * len(cpws)]
        idx_vs = refs[2 * len(cpws):3 * len(cpws)]
        rows_v = refs[3 * len(cpws)]
        sem = refs[3 * len(cpws) + 1]
        w = lax.axis_index("s") * _NC + lax.axis_index("c")
        for ids, out, idx_v, cpw in zip(ids_refs, out_refs, idx_vs, cpws):
            pltpu.sync_copy(ids.at[w], idx_v)
            nsc = -(-cpw * _CH // buf_rows)
            npsc = cpw // nsc
            for s in range(nsc):
                hs = [pltpu.async_copy(emb.at[idx_v.at[s * npsc + j]],
                                       rows_v.at[pl.ds(j * _CH, _CH)], sem)
                      for j in range(npsc)]
                for h in hs:
                    h.wait()
                rows = npsc * _CH
                pltpu.sync_copy(
                    rows_v.at[pl.ds(0, rows)],
                    out.at[pl.ds(w * cpw * _CH + s * rows, rows)])

    return gather(emb2, *idlists)


def _selu(v):
    return 1.0507009873554805 * jnp.where(
        v > 0, v, 1.6732632423543772 * (jnp.exp(v) - 1.0))


def _softplus(z):
    return jnp.maximum(z, 0.0) + jnp.log(1.0 + jnp.exp(-jnp.abs(z)))


def _half(rows, m, D):
    return jnp.where(m > 0.5, rows[:, D:], rows[:, :D])


def _tc_loss_body(K, D, cr, xr, nr, cm, xm, nm, wc, bc, wx, bx, out):
    i = pl.program_id(0)
    bt = cr.shape[0]
    dn = (((1,), (1,)), ((), ()))
    c = _selu(lax.dot_general(_half(cr[...], cm[...], D), wc[...], dn,
                              preferred_element_type=jnp.float32) + bc[...])
    x = _selu(lax.dot_general(_half(xr[...], xm[...], D), wx[...], dn,
                              preferred_element_type=jnp.float32) + bx[...])
    n = _selu(lax.dot_general(_half(nr[...], nm[...], D), wx[...], dn,
                              preferred_element_type=jnp.float32) + bx[...])
    pos = jnp.sum(c * x, axis=1, keepdims=True)
    pos = jnp.clip(pos, -10.0, 10.0)
    total = jnp.sum(_softplus(-pos))
    for k in range(K):
        nk = n[k * bt:(k + 1) * bt, :]
        neg = jnp.sum(nk * c, axis=1, keepdims=True)
        neg = jnp.clip(neg, -10.0, 10.0)
        total += jnp.sum(_softplus(neg))

    @pl.when(i == 0)
    def _():
        out[0, 0] = 0.0

    out[0, 0] += total


def _tc_loss(crows, xrows, nrows, cm, xm, nm, wc, bc, wx, bx, B, K, D):
    grid = (B // _BT,)
    return pl.pallas_call(
        functools.partial(_tc_loss_body, K, D),
        grid=grid,
        in_specs=[
            pl.BlockSpec((_BT, 2 * D), lambda i: (i, 0)),
            pl.BlockSpec((_BT, 2 * D), lambda i: (i, 0)),
            pl.BlockSpec((_BT * K, 2 * D), lambda i: (i, 0)),
            pl.BlockSpec((_BT, 1), lambda i: (i, 0)),
            pl.BlockSpec((_BT, 1), lambda i: (i, 0)),
            pl.BlockSpec((_BT * K, 1), lambda i: (i, 0)),
            pl.BlockSpec((D, D), lambda i: (0, 0)),
            pl.BlockSpec((1, D), lambda i: (0, 0)),
            pl.BlockSpec((D, D), lambda i: (0, 0)),
            pl.BlockSpec((1, D), lambda i: (0, 0)),
        ],
        out_specs=pl.BlockSpec((1, 1), lambda i: (0, 0),
                               memory_space=pltpu.SMEM),
        out_shape=jax.ShapeDtypeStruct((1, 1), jnp.float32),
        compiler_params=pltpu.CompilerParams(
            dimension_semantics=("arbitrary",)),
    )(crows, xrows, nrows, cm, xm, nm, wc, bc, wx, bx)


def _slice_ids(ids):
    pid = (ids // _VC) * _HC + (ids & (_HC - 1))
    m = ((ids // _HC) & 1).astype(jnp.float32)
    return pid, m


def kernel(center_ids, context_ids, neg_context_ids, center_emb, context_emb,
           W_center, b_center, W_context, b_context):
    B = center_ids.shape[0]
    K = neg_context_ids.shape[1]
    V, D = center_emb.shape
    cids = center_ids.astype(jnp.int32)
    xids = context_ids.astype(jnp.int32)
    nids = (neg_context_ids.astype(jnp.int32)
            .reshape(B // _BT, _BT, K).transpose(0, 2, 1).reshape(B * K))
    cpid, cm = _slice_ids(cids)
    xpid, xm = _slice_ids(xids)
    npid, nm = _slice_ids(nids)
    cpw = B // _NW // _CH
    npw = B * K // _NW // _CH
    xemb2 = _repack(context_emb.T, V, D)
    xrows, nrows = _sc_gather(
        xemb2,
        [xpid.reshape(_NW, cpw, _CH), npid.reshape(_NW, npw, _CH)], D)
    cemb2 = _repack(center_emb.T, V, D)
    (crows,) = _sc_gather(cemb2, [cpid.reshape(_NW, cpw, _CH)], D)
    total = _tc_loss(crows, xrows, nrows,
                     cm.reshape(B, 1), xm.reshape(B, 1),
                     nm.reshape(B * K, 1),
                     W_center, b_center.reshape(1, D),
                     W_context, b_context.reshape(1, D), B, K, D)
    return total[0, 0] / B


# speedup vs baseline: 1.0080x; 1.0080x over previous
import functools

import jax
import jax.numpy as jnp
from jax import lax
from jax.experimental import pallas as pl
from jax.experimental.pallas import tpu as pltpu
from jax.experimental.pallas import tpu_sc as plsc

_NC = 2
_NS = 16
_NW = _NC * _NS
_CH = 128
_VC = 32768
_HC = _VC // 2
_BT = 1024


def _repack_body(tT, out):
    x = tT[...]
    x2 = jnp.concatenate([x[:, :_HC], x[:, _HC:]], axis=0)
    out[...] = jnp.transpose(x2)


def _repack(tableT, V, D):
    nb = pl.cdiv(V, _VC)
    return pl.pallas_call(
        _repack_body,
        grid=(nb,),
        in_specs=[pl.BlockSpec((D, _VC), lambda i: (0, i))],
        out_specs=pl.BlockSpec((_HC, 2 * D), lambda i: (i, 0)),
        out_shape=jax.ShapeDtypeStruct((nb * _HC, 2 * D), jnp.float32),
        compiler_params=pltpu.CompilerParams(
            dimension_semantics=("parallel",)),
    )(tableT)


def _sc_gather(emb2, idlists, D):
    W = 2 * D
    cpws = [ids.shape[1] for ids in idlists]
    buf_rows = 640

    mesh = plsc.VectorSubcoreMesh(core_axis_name="c", subcore_axis_name="s")

    @functools.partial(
        pl.kernel,
        mesh=mesh,
        out_type=[
            jax.ShapeDtypeStruct((_NW * cpw * _CH, W), jnp.float32)
            for cpw in cpws
        ],
        scratch_types=[
            *[pltpu.VMEM((cpw, _CH), jnp.int32) for cpw in cpws],
            pltpu.VMEM((buf_rows, W), jnp.float32),
            pltpu.SemaphoreType.DMA,
        ],
    )
    def gather(emb, *refs):
        ids_refs = refs[:len(cpws)]
        out_refs = refs[len(cpws):2 * len(cpws)]
        idx_vs = refs[2 * len(cpws):3 * len(cpws)]
        rows_v = refs[3 * len(cpws)]
        sem = refs[3 * len(cpws) + 1]
        w = lax.axis_index("s") * _NC + lax.axis_index("c")
        for ids, out, idx_v, cpw in zip(ids_refs, out_refs, idx_vs, cpws):
            pltpu.sync_copy(ids.at[w], idx_v)
            nsc = -(-cpw * _CH // buf_rows)
            npsc = cpw // nsc
            for s in range(nsc):
                hs = [pltpu.async_copy(emb.at[idx_v.at[s * npsc + j]],
                                       rows_v.at[pl.ds(j * _CH, _CH)], sem)
                      for j in range(npsc)]
                for h in hs:
                    h.wait()
                rows = npsc * _CH
                pltpu.sync_copy(
                    rows_v.at[pl.ds(0, rows)],
                    out.at[pl.ds(w * cpw * _CH + s * rows, rows)])

    return gather(emb2, *idlists)


def _selu(v):
    return 1.0507009873554805 * jnp.where(
        v > 0, v, 1.6732632423543772 * (jnp.exp(v) - 1.0))


def _softplus(z):
    return jnp.maximum(z, 0.0) + jnp.log(1.0 + jnp.exp(-jnp.abs(z)))


def _half(rows, m, D):
    return jnp.where(m > 0.5, rows[:, D:], rows[:, :D])


def _tc_loss_body(K, D, cr, xr, nr, cm, xm, nm, wc, bc, wx, bx, out):
    i = pl.program_id(0)
    bt = cr.shape[0]
    dn = (((1,), (1,)), ((), ()))
    c = _selu(lax.dot_general(_half(cr[...], cm[...], D), wc[...], dn,
                              preferred_element_type=jnp.float32) + bc[...])
    x = _selu(lax.dot_general(_half(xr[...], xm[...], D), wx[...], dn,
                              preferred_element_type=jnp.float32) + bx[...])
    n = _selu(lax.dot_general(_half(nr[...], nm[...], D), wx[...], dn,
                              preferred_element_type=jnp.float32) + bx[...])
    pos = jnp.sum(c * x, axis=1, keepdims=True)
    pos = jnp.clip(pos, -10.0, 10.0)
    total = jnp.sum(_softplus(-pos))
    for k in range(K):
        nk = n[k * bt:(k + 1) * bt, :]
        neg = jnp.sum(nk * c, axis=1, keepdims=True)
        neg = jnp.clip(neg, -10.0, 10.0)
        total += jnp.sum(_softplus(neg))

    @pl.when(i == 0)
    def _():
        out[0, 0] = 0.0

    out[0, 0] += total


def _tc_loss(crows, xrows, nrows, cm, xm, nm, wc, bc, wx, bx, B, K, D):
    grid = (B // _BT,)
    return pl.pallas_call(
        functools.partial(_tc_loss_body, K, D),
        grid=grid,
        in_specs=[
            pl.BlockSpec((_BT, 2 * D), lambda i: (i, 0)),
            pl.BlockSpec((_BT, 2 * D), lambda i: (i, 0)),
            pl.BlockSpec((_BT * K, 2 * D), lambda i: (i, 0)),
            pl.BlockSpec((_BT, 1), lambda i: (i, 0)),
            pl.BlockSpec((_BT, 1), lambda i: (i, 0)),
            pl.BlockSpec((_BT * K, 1), lambda i: (i, 0)),
            pl.BlockSpec((D, D), lambda i: (0, 0)),
            pl.BlockSpec((1, D), lambda i: (0, 0)),
            pl.BlockSpec((D, D), lambda i: (0, 0)),
            pl.BlockSpec((1, D), lambda i: (0, 0)),
        ],
        out_specs=pl.BlockSpec((1, 1), lambda i: (0, 0),
                               memory_space=pltpu.SMEM),
        out_shape=jax.ShapeDtypeStruct((1, 1), jnp.float32),
        compiler_params=pltpu.CompilerParams(
            dimension_semantics=("arbitrary",)),
    )(crows, xrows, nrows, cm, xm, nm, wc, bc, wx, bx)


def _slice_ids(ids):
    pid = (ids // _VC) * _HC + (ids & (_HC - 1))
    m = ((ids // _HC) & 1).astype(jnp.float32)
    return pid, m


def kernel(center_ids, context_ids, neg_context_ids, center_emb, context_emb,
           W_center, b_center, W_context, b_context):
    B = center_ids.shape[0]
    K = neg_context_ids.shape[1]
    V, D = center_emb.shape
    cids = center_ids.astype(jnp.int32)
    xids = context_ids.astype(jnp.int32)
    nids = (neg_context_ids.astype(jnp.int32)
            .reshape(B // _BT, _BT, K).transpose(0, 2, 1).reshape(B * K))
    cpid, cm = _slice_ids(cids)
    xpid, xm = _slice_ids(xids)
    npid, nm = _slice_ids(nids)
    cpw = B // _NW // _CH
    npw = B * K // _NW // _CH
    xemb2 = _repack(context_emb.T, V, D)
    xrows, nrows = _sc_gather(
        xemb2,
        [xpid.reshape(_NW, cpw, _CH), npid.reshape(_NW, npw, _CH)], D)
    cemb2 = _repack(center_emb.T, V, D)
    (crows,) = _sc_gather(cemb2, [cpid.reshape(_NW, cpw, _CH)], D)
    total = _tc_loss(crows, xrows, nrows,
                     cm.reshape(B, 1), xm.reshape(B, 1),
                     nm.reshape(B * K, 1),
                     W_center, b_center.reshape(1, D),
                     W_context, b_context.reshape(1, D), B, K, D)
    return total[0, 0] / B
